# baseline (speedup 1.0000x reference)
import functools

import jax
import jax.numpy as jnp
import numpy as np
from jax import lax
from jax.experimental import pallas as pl
from jax.experimental.pallas import tpu as pltpu
from jax.experimental.pallas import tpu_sc as plsc

N_NODES = 10000
D_FEAT = 128
N_EDGES = 320000

NC = 2
NS = 16
NW = NC * NS
CH = 128
NCH = 80
E_PAD = NW * NCH * CH
N_PAD = 10240
RPT = N_PAD // NS

_STORE_ORDER = np.empty(D_FEAT, np.int32)
for _k in range(D_FEAT // 32):
    for _i in range(16):
        _STORE_ORDER[32 * _k + 2 * _i] = 32 * _k + _i
        _STORE_ORDER[32 * _k + 2 * _i + 1] = 32 * _k + 16 + _i


def _sc_aggregate(nfb, src, dst, w):
    mesh = plsc.VectorSubcoreMesh(core_axis_name="c", subcore_axis_name="s")

    @functools.partial(
        pl.kernel,
        mesh=mesh,
        out_type=jax.ShapeDtypeStruct((NC, N_PAD, D_FEAT), jnp.float32),
        compiler_params=pltpu.CompilerParams(use_tc_tiling_on_sc=False,
                                             needs_layout_passes=False),
        scratch_types=[
            pltpu.VMEM_SHARED((N_PAD, D_FEAT), jnp.float32),
            pltpu.VMEM((NCH, CH), jnp.int32),
            pltpu.VMEM((NCH, CH), jnp.int32),
            pltpu.VMEM((CH,), jnp.float32),
            pltpu.VMEM((CH, D_FEAT // 2), jnp.float32),
            pltpu.VMEM((CH, D_FEAT), jnp.float32),
            pltpu.SemaphoreType.DMA,
            pltpu.SemaphoreType.DMA,
        ],
    )
    def body(nfb_hbm, src_hbm, dst_hbm, w_hbm, out_hbm, acc,
             sidx, didx, wvb, rows_bf, rows_f, semw, semg):
        c = lax.axis_index("c")
        s = lax.axis_index("s")
        wid = c * NS + s

        pltpu.sync_copy(src_hbm.at[wid], sidx)
        pltpu.sync_copy(dst_hbm.at[wid], didx)

        zero = jnp.zeros((16,), jnp.float32)

        def zrow(r, carry):
            for k in range(D_FEAT // 16):
                rows_f[r, pl.ds(k * 16, 16)] = zero
            return carry

        lax.fori_loop(0, CH, zrow, 0)
        for j in range(RPT // CH):
            pltpu.sync_copy(rows_f, acc.at[pl.ds(s * RPT + j * CH, CH)])
        plsc.subcore_barrier()

        dnums = lax.GatherDimensionNumbers(
            offset_dims=(), collapsed_slice_dims=(0,),
            start_index_map=(0,))

        def chunk(ci, carry):
            pltpu.async_copy(w_hbm.at[wid, pl.ds(ci * CH, CH)], wvb, semw)
            pltpu.async_copy(nfb_hbm.at[sidx.at[ci]], rows_bf, semg).wait()
            pltpu.make_async_copy(
                w_hbm.at[wid, pl.ds(ci * CH, CH)], wvb, semw).wait()

            def grp(g, inner):
                w16 = wvb[pl.ds(g * 16, 16)]
                for j in range(16):
                    sp = lax.gather(
                        w16, jnp.full((16, 1), j, jnp.int32), dnums,
                        slice_sizes=(1,),
                        mode=lax.GatherScatterMode.PROMISE_IN_BOUNDS)
                    r = g * 16 + j
                    for k in range(D_FEAT // 32):
                        ab = plsc.bitcast(rows_bf[r, pl.ds(k * 16, 16)],
                                          jnp.bfloat16)
                        a, b = plsc.unpack(
                            ab, format=plsc.PackFormat.INTERLEAVED)
                        rows_f[r, pl.ds(k * 32, 16)] = (
                            a.astype(jnp.float32) * sp)
                        rows_f[r, pl.ds(k * 32 + 16, 16)] = (
                            b.astype(jnp.float32) * sp)
                return inner

            lax.fori_loop(0, CH // 16, grp, 0)
            pltpu.sync_copy(rows_f, acc.at[didx.at[ci]], add=True)
            return carry

        lax.fori_loop(0, NCH, chunk, 0)
        plsc.subcore_barrier()

        pltpu.sync_copy(acc.at[pl.ds(s * RPT, RPT)],
                        out_hbm.at[c, pl.ds(s * RPT, RPT)])

    return body(nfb, src, dst, w)


def _tc_body(p_ref, m_ref, b_ref, o_ref):
    agg = p_ref[0] + p_ref[1]
    o_ref[...] = jnp.dot(agg, m_ref[...],
                         preferred_element_type=jnp.float32,
                         precision=lax.Precision.HIGHEST) + b_ref[...]


def kernel(n_feat, edge_index, edge_weights, W_in, b_in):
    src = edge_index[0].astype(jnp.int32)
    dst = edge_index[1].astype(jnp.int32)
    w = edge_weights.reshape(-1).astype(jnp.float32)

    pad = E_PAD - N_EDGES
    src = jnp.concatenate([src, jnp.zeros((pad,), jnp.int32)])
    dst = jnp.concatenate([dst, jnp.zeros((pad,), jnp.int32)])
    w = jnp.concatenate([w, jnp.zeros((pad,), jnp.float32)])
    src = src.reshape(NW, NCH, CH)
    dst = dst.reshape(NW, NCH, CH)
    w = w.reshape(NW, NCH * CH)

    nfb = lax.bitcast_convert_type(
        n_feat[:, _STORE_ORDER].astype(jnp.bfloat16).reshape(
            N_NODES, D_FEAT // 2, 2),
        jnp.float32)

    partials = _sc_aggregate(nfb, src, dst, w)[:, :N_NODES, :]

    m = W_in.T + jnp.eye(D_FEAT, dtype=jnp.float32)
    out = pl.pallas_call(
        _tc_body,
        out_shape=jax.ShapeDtypeStruct((N_NODES, D_FEAT), jnp.float32),
    )(partials, m, b_in.reshape(1, D_FEAT))
    return out


# speedup vs baseline: 1.0003x; 1.0003x over previous
import functools

import jax
import jax.numpy as jnp
import numpy as np
from jax import lax
from jax.experimental import pallas as pl
from jax.experimental.pallas import tpu as pltpu
from jax.experimental.pallas import tpu_sc as plsc

N_NODES = 10000
D_FEAT = 128
N_EDGES = 320000

NC = 2
NS = 16
NW = NC * NS
CH = 128
NCH = 80
E_PAD = NW * NCH * CH
N_PAD = 10240
RPT = N_PAD // NS

_STORE_ORDER = np.empty(D_FEAT, np.int32)
for _k in range(D_FEAT // 32):
    for _i in range(16):
        _STORE_ORDER[32 * _k + 2 * _i] = 32 * _k + _i
        _STORE_ORDER[32 * _k + 2 * _i + 1] = 32 * _k + 16 + _i


def _sc_aggregate(nfb, src, dst, w):
    mesh = plsc.VectorSubcoreMesh(core_axis_name="c", subcore_axis_name="s")

    @functools.partial(
        pl.kernel,
        mesh=mesh,
        out_type=jax.ShapeDtypeStruct((NC, N_PAD, D_FEAT), jnp.float32),
        compiler_params=pltpu.CompilerParams(use_tc_tiling_on_sc=False,
                                             needs_layout_passes=False),
        scratch_types=[
            pltpu.VMEM_SHARED((N_PAD, D_FEAT), jnp.float32),
            pltpu.VMEM((NCH, CH), jnp.int32),
            pltpu.VMEM((NCH, CH), jnp.int32),
            pltpu.VMEM((CH,), jnp.float32),
            pltpu.VMEM((CH, D_FEAT // 2), jnp.float32),
            pltpu.VMEM((CH, D_FEAT), jnp.float32),
            pltpu.SemaphoreType.DMA,
            pltpu.SemaphoreType.DMA,
        ],
    )
    def body(nfb_hbm, src_hbm, dst_hbm, w_hbm, out_hbm, acc,
             sidx, didx, wvb, rows_bf, rows_f, semw, semg):
        c = lax.axis_index("c")
        s = lax.axis_index("s")
        wid = c * NS + s

        pltpu.sync_copy(src_hbm.at[wid], sidx)
        pltpu.sync_copy(dst_hbm.at[wid], didx)

        zero = jnp.zeros((16,), jnp.float32)

        def zrow(r, carry):
            for k in range(D_FEAT // 16):
                rows_f[r, pl.ds(k * 16, 16)] = zero
            return carry

        lax.fori_loop(0, CH, zrow, 0)
        for j in range(RPT // CH):
            pltpu.sync_copy(rows_f, acc.at[pl.ds(s * RPT + j * CH, CH)])
        plsc.subcore_barrier()

        dnums = lax.GatherDimensionNumbers(
            offset_dims=(), collapsed_slice_dims=(0,),
            start_index_map=(0,))

        def chunk(ci, carry):
            pltpu.async_copy(w_hbm.at[wid, pl.ds(ci * CH, CH)], wvb, semw)
            pltpu.async_copy(nfb_hbm.at[sidx.at[ci]], rows_bf, semg).wait()
            pltpu.make_async_copy(
                w_hbm.at[wid, pl.ds(ci * CH, CH)], wvb, semw).wait()

            def grp(g, inner):
                w16 = wvb[pl.ds(g * 16, 16)]
                for j in range(16):
                    sp = lax.gather(
                        w16, jnp.full((16, 1), j, jnp.int32), dnums,
                        slice_sizes=(1,),
                        mode=lax.GatherScatterMode.PROMISE_IN_BOUNDS)
                    r = g * 16 + j
                    for k in range(D_FEAT // 32):
                        wi = plsc.bitcast(rows_bf[r, pl.ds(k * 16, 16)],
                                          jnp.int32)
                        a = plsc.bitcast(wi << 16, jnp.float32)
                        b = plsc.bitcast(wi & jnp.int32(-65536), jnp.float32)
                        rows_f[r, pl.ds(k * 32, 16)] = a * sp
                        rows_f[r, pl.ds(k * 32 + 16, 16)] = b * sp
                return inner

            lax.fori_loop(0, CH // 16, grp, 0)
            pltpu.sync_copy(rows_f, acc.at[didx.at[ci]], add=True)
            return carry

        lax.fori_loop(0, NCH, chunk, 0)
        plsc.subcore_barrier()

        pltpu.sync_copy(acc.at[pl.ds(s * RPT, RPT)],
                        out_hbm.at[c, pl.ds(s * RPT, RPT)])

    return body(nfb, src, dst, w)


def _tc_body(p_ref, m_ref, b_ref, o_ref):
    agg = p_ref[0] + p_ref[1]
    o_ref[...] = jnp.dot(agg, m_ref[...],
                         preferred_element_type=jnp.float32,
                         precision=lax.Precision.HIGHEST) + b_ref[...]


def kernel(n_feat, edge_index, edge_weights, W_in, b_in):
    src = edge_index[0].astype(jnp.int32)
    dst = edge_index[1].astype(jnp.int32)
    w = edge_weights.reshape(-1).astype(jnp.float32)

    pad = E_PAD - N_EDGES
    src = jnp.concatenate([src, jnp.zeros((pad,), jnp.int32)])
    dst = jnp.concatenate([dst, jnp.zeros((pad,), jnp.int32)])
    w = jnp.concatenate([w, jnp.zeros((pad,), jnp.float32)])
    src = src.reshape(NW, NCH, CH)
    dst = dst.reshape(NW, NCH, CH)
    w = w.reshape(NW, NCH * CH)

    nfb = lax.bitcast_convert_type(
        n_feat[:, _STORE_ORDER].astype(jnp.bfloat16).reshape(
            N_NODES, D_FEAT // 2, 2),
        jnp.float32)

    partials = _sc_aggregate(nfb, src, dst, w)[:, :N_NODES, :]

    m = W_in.T + jnp.eye(D_FEAT, dtype=jnp.float32)
    out = pl.pallas_call(
        _tc_body,
        out_shape=jax.ShapeDtypeStruct((N_NODES, D_FEAT), jnp.float32),
    )(partials, m, b_in.reshape(1, D_FEAT))
    return out


# speedup vs baseline: 1.4165x; 1.4161x over previous
import functools

import jax
import jax.numpy as jnp
import numpy as np
from jax import lax
from jax.experimental import pallas as pl
from jax.experimental.pallas import tpu as pltpu
from jax.experimental.pallas import tpu_sc as plsc

N_NODES = 10000
D_FEAT = 128
N_EDGES = 320000

NC = 2
NS = 16
NW = NC * NS
DW = D_FEAT // 2
CH = 128
NCH = 80
E_PAD = NW * NCH * CH
N_PAD = 10240
RPT = N_PAD // NS

_STORE_ORDER = np.empty(D_FEAT, np.int32)
for _k in range(D_FEAT // 32):
    for _i in range(16):
        _STORE_ORDER[32 * _k + 2 * _i] = 32 * _k + _i
        _STORE_ORDER[32 * _k + 2 * _i + 1] = 32 * _k + 16 + _i


def _sc_aggregate(nfb, src, dst, w):
    mesh = plsc.VectorSubcoreMesh(core_axis_name="c", subcore_axis_name="s")

    @functools.partial(
        pl.kernel,
        mesh=mesh,
        out_type=jax.ShapeDtypeStruct((NC, N_PAD, D_FEAT), jnp.float32),
        compiler_params=pltpu.CompilerParams(use_tc_tiling_on_sc=False,
                                             needs_layout_passes=False),
        scratch_types=[
            pltpu.VMEM_SHARED((N_PAD, D_FEAT), jnp.float32),
            pltpu.VMEM((CH,), jnp.int32),
            pltpu.VMEM((CH,), jnp.int32),
            pltpu.VMEM((CH,), jnp.int32),
            pltpu.VMEM((CH,), jnp.int32),
            pltpu.VMEM((CH,), jnp.float32),
            pltpu.VMEM((CH,), jnp.float32),
            pltpu.VMEM((CH, DW), jnp.float32),
            pltpu.VMEM((CH, DW), jnp.float32),
            pltpu.VMEM((CH, D_FEAT), jnp.float32),
            pltpu.SemaphoreType.DMA,
            pltpu.SemaphoreType.DMA,
            pltpu.SemaphoreType.DMA,
            pltpu.SemaphoreType.DMA,
        ],
    )
    def body(nfb_hbm, src_hbm, dst_hbm, w_hbm, out_hbm, acc,
             sid0, sid1, did0, did1, wv0, wv1, rowsb0, rowsb1, rows_f,
             semi0, semi1, semg0, semg1):
        c = lax.axis_index("c")
        s = lax.axis_index("s")
        wid = c * NS + s

        sid = (sid0, sid1)
        did = (did0, did1)
        wv = (wv0, wv1)
        rowsb = (rowsb0, rowsb1)
        semi = (semi0, semi1)
        semg = (semg0, semg1)

        def start_idx(ci, p):
            off = ci * CH
            pltpu.async_copy(src_hbm.at[wid, pl.ds(off, CH)], sid[p], semi[p])
            pltpu.async_copy(dst_hbm.at[wid, pl.ds(off, CH)], did[p], semi[p])
            pltpu.async_copy(w_hbm.at[wid, pl.ds(off, CH)], wv[p], semi[p])

        def wait_idx(ci, p):
            off = ci * CH
            pltpu.make_async_copy(
                src_hbm.at[wid, pl.ds(off, CH)], sid[p], semi[p]).wait()
            pltpu.make_async_copy(
                dst_hbm.at[wid, pl.ds(off, CH)], did[p], semi[p]).wait()
            pltpu.make_async_copy(
                w_hbm.at[wid, pl.ds(off, CH)], wv[p], semi[p]).wait()

        zero = jnp.zeros((16,), jnp.float32)

        def zrow(r, carry):
            for k in range(D_FEAT // 16):
                rows_f[r, pl.ds(k * 16, 16)] = zero
            return carry

        lax.fori_loop(0, CH, zrow, 0)
        for j in range(RPT // CH):
            pltpu.sync_copy(rows_f, acc.at[pl.ds(s * RPT + j * CH, CH)])
        plsc.subcore_barrier()

        dnums = lax.GatherDimensionNumbers(
            offset_dims=(), collapsed_slice_dims=(0,),
            start_index_map=(0,))

        def scale(p):
            def grp(g, inner):
                w16 = wv[p][pl.ds(g * 16, 16)]
                for j in range(16):
                    sp = lax.gather(
                        w16, jnp.full((16, 1), j, jnp.int32), dnums,
                        slice_sizes=(1,),
                        mode=lax.GatherScatterMode.PROMISE_IN_BOUNDS)
                    r = g * 16 + j
                    for k in range(D_FEAT // 32):
                        wi = plsc.bitcast(rowsb[p][r, pl.ds(k * 16, 16)],
                                          jnp.int32)
                        a = plsc.bitcast(wi << 16, jnp.float32)
                        b = plsc.bitcast(wi & jnp.int32(-65536), jnp.float32)
                        rows_f[r, pl.ds(k * 32, 16)] = a * sp
                        rows_f[r, pl.ds(k * 32 + 16, 16)] = b * sp
                return inner

            lax.fori_loop(0, CH // 16, grp, 0)

        NPAIR = NCH // 2
        start_idx(0, 0)
        start_idx(1, 1)
        wait_idx(0, 0)
        pltpu.async_copy(nfb_hbm.at[sid[0]], rowsb[0], semg[0])

        def step(ci, p, po):
            if p == 0:
                wait_idx(ci + 1, 1)
                pltpu.async_copy(nfb_hbm.at[sid[1]], rowsb[1], semg[1])
            else:
                @pl.when(po != NPAIR - 1)
                def _():
                    wait_idx(ci + 1, 0)
                    pltpu.async_copy(nfb_hbm.at[sid[0]], rowsb[0], semg[0])

            pltpu.make_async_copy(nfb_hbm.at[sid[p]], rowsb[p], semg[p]).wait()
            scale(p)
            pltpu.sync_copy(rows_f, acc.at[did[p]], add=True)

            @pl.when(po != NPAIR - 1)
            def _():
                start_idx(ci + 2, p)

        def pair(po, carry):
            step(po * 2, 0, po)
            step(po * 2 + 1, 1, po)
            return carry

        lax.fori_loop(0, NPAIR, pair, 0)
        plsc.subcore_barrier()

        pltpu.sync_copy(acc.at[pl.ds(s * RPT, RPT)],
                        out_hbm.at[c, pl.ds(s * RPT, RPT)])

    return body(nfb, src, dst, w)


def _tc_body(p_ref, m_ref, b_ref, o_ref):
    agg = p_ref[0] + p_ref[1]
    o_ref[...] = jnp.dot(agg, m_ref[...],
                         preferred_element_type=jnp.float32,
                         precision=lax.Precision.HIGHEST) + b_ref[...]


def kernel(n_feat, edge_index, edge_weights, W_in, b_in):
    src = edge_index[0].astype(jnp.int32)
    dst = edge_index[1].astype(jnp.int32)
    w = edge_weights.reshape(-1).astype(jnp.float32)

    pad = E_PAD - N_EDGES
    src = jnp.concatenate([src, jnp.zeros((pad,), jnp.int32)])
    dst = jnp.concatenate([dst, jnp.zeros((pad,), jnp.int32)])
    w = jnp.concatenate([w, jnp.zeros((pad,), jnp.float32)])
    src = src.reshape(NW, NCH * CH)
    dst = dst.reshape(NW, NCH * CH)
    w = w.reshape(NW, NCH * CH)

    nfb = lax.bitcast_convert_type(
        n_feat[:, _STORE_ORDER].astype(jnp.bfloat16).reshape(N_NODES, DW, 2),
        jnp.float32)

    partials = _sc_aggregate(nfb, src, dst, w)[:, :N_NODES, :]

    m = W_in.T + jnp.eye(D_FEAT, dtype=jnp.float32)
    out = pl.pallas_call(
        _tc_body,
        out_shape=jax.ShapeDtypeStruct((N_NODES, D_FEAT), jnp.float32),
    )(partials, m, b_in.reshape(1, D_FEAT))
    return out


# speedup vs baseline: 1.5400x; 1.0872x over previous
import functools

import jax
import jax.numpy as jnp
from jax import lax
from jax.experimental import pallas as pl
from jax.experimental.pallas import tpu as pltpu
from jax.experimental.pallas import tpu_sc as plsc

N_NODES = 10000
D_FEAT = 128
N_EDGES = 320000

NC = 2
NS = 16
NW = NC * NS
CH = 128
NCH = 79
E_PAD = NW * NCH * CH
N_PAD = 10240
RPT = N_PAD // NS


def _sc_aggregate(n_feat, src, dst, w):
    mesh = plsc.VectorSubcoreMesh(core_axis_name="c", subcore_axis_name="s")

    @functools.partial(
        pl.kernel,
        mesh=mesh,
        out_type=jax.ShapeDtypeStruct((NC, N_PAD, D_FEAT), jnp.float32),
        scratch_types=[
            pltpu.VMEM_SHARED((N_PAD, D_FEAT), jnp.float32),
            pltpu.VMEM((NCH, CH), jnp.int32),
            pltpu.VMEM((NCH, CH), jnp.int32),
            pltpu.VMEM((NCH, CH), jnp.float32),
            pltpu.VMEM((CH, D_FEAT), jnp.float32),
            pltpu.SemaphoreType.DMA,
        ],
    )
    def body(nf_hbm, src_hbm, dst_hbm, w_hbm, out_hbm, acc, sidx, didx, wv,
             rows, sem):
        c = lax.axis_index("c")
        s = lax.axis_index("s")
        wid = c * NS + s

        pltpu.sync_copy(src_hbm.at[wid], sidx)
        pltpu.sync_copy(dst_hbm.at[wid], didx)
        pltpu.sync_copy(w_hbm.at[wid], wv)

        zero = jnp.zeros((16,), jnp.float32)

        def zrow(r, carry):
            for k in range(D_FEAT // 16):
                rows[r, pl.ds(k * 16, 16)] = zero
            return carry

        lax.fori_loop(0, CH, zrow, 0)
        for j in range(RPT // CH):
            pltpu.sync_copy(rows, acc.at[pl.ds(s * RPT + j * CH, CH)])
        plsc.subcore_barrier()

        def chunk(ci, carry):
            pltpu.async_copy(nf_hbm.at[sidx.at[ci]], rows, sem).wait()

            dnums = lax.GatherDimensionNumbers(
                offset_dims=(), collapsed_slice_dims=(0,),
                start_index_map=(0,))

            def grp(g, inner):
                w16 = wv[ci, pl.ds(g * 16, 16)]
                for j in range(16):
                    sp = lax.gather(
                        w16, jnp.full((16, 1), j, jnp.int32), dnums,
                        slice_sizes=(1,),
                        mode=lax.GatherScatterMode.PROMISE_IN_BOUNDS)
                    r = g * 16 + j
                    for k in range(D_FEAT // 16):
                        rows[r, pl.ds(k * 16, 16)] = (
                            rows[r, pl.ds(k * 16, 16)] * sp)
                return inner

            lax.fori_loop(0, CH // 16, grp, 0)
            pltpu.sync_copy(rows, acc.at[didx.at[ci]], add=True)
            return carry

        lax.fori_loop(0, NCH, chunk, 0)
        plsc.subcore_barrier()

        pltpu.sync_copy(acc.at[pl.ds(s * RPT, RPT)],
                        out_hbm.at[c, pl.ds(s * RPT, RPT)])

    return body(n_feat, src, dst, w)


def _tc_body(p_ref, m_ref, b_ref, o_ref):
    agg = p_ref[0] + p_ref[1]
    o_ref[...] = jnp.dot(agg, m_ref[...],
                         preferred_element_type=jnp.float32,
                         precision=lax.Precision.HIGHEST) + b_ref[...]


def kernel(n_feat, edge_index, edge_weights, W_in, b_in):
    src = edge_index[0].astype(jnp.int32)
    dst = edge_index[1].astype(jnp.int32)
    w = edge_weights.reshape(-1).astype(jnp.float32)

    pad = E_PAD - N_EDGES
    src = jnp.concatenate([src, jnp.zeros((pad,), jnp.int32)])
    dst = jnp.concatenate([dst, jnp.zeros((pad,), jnp.int32)])
    w = jnp.concatenate([w, jnp.zeros((pad,), jnp.float32)])
    src = src.reshape(NW, NCH, CH)
    dst = dst.reshape(NW, NCH, CH)
    w = w.reshape(NW, NCH, CH)

    partials = _sc_aggregate(n_feat, src, dst, w)[:, :N_NODES, :]

    m = W_in.T + jnp.eye(D_FEAT, dtype=jnp.float32)
    out = pl.pallas_call(
        _tc_body,
        out_shape=jax.ShapeDtypeStruct((N_NODES, D_FEAT), jnp.float32),
    )(partials, m, b_in.reshape(1, D_FEAT))
    return out
